# baseline (speedup 1.0000x reference)
import functools

import jax
import jax.numpy as jnp
from jax import lax
from jax.experimental import pallas as pl
from jax.experimental.pallas import tpu as pltpu
from jax.experimental.pallas import tpu_sc as plsc

N = 5000
E = 20000
FIN = 14
D = 64
DW = 128
B = 128
STEPS = 6

N_PAD = 5120
E_PAD = 20480
CH = 128

F32 = jnp.float32


def _lrelu(v):
    return jnp.where(v >= 0, v, 0.01 * v)


def _lin0_body(x_ref, w_ref, b_ref, o_ref):
    o_ref[:, 0:D] = _lrelu(
        jnp.dot(x_ref[:], w_ref[:], preferred_element_type=F32) + b_ref[:])
    o_ref[:, D:DW] = jnp.zeros((N_PAD, DW - D), F32)


def _lin0(x_pad, w0t, b0):
    return pl.pallas_call(
        _lin0_body,
        out_shape=jax.ShapeDtypeStruct((N_PAD, DW), F32),
    )(x_pad, w0t, b0.reshape(1, D))


_EW_BLK = 512


def _ew_body(ea_ref, a1t_ref, c1_ref, a2t_ref, c2_ref, o_ref):
    pid = pl.program_id(0)
    hid = _lrelu(
        jnp.dot(ea_ref[:], a1t_ref[:], preferred_element_type=F32) + c1_ref[:])
    val = jnp.dot(hid, a2t_ref[:], preferred_element_type=F32) + c2_ref[:]
    eid = pid * _EW_BLK + lax.broadcasted_iota(jnp.int32, (_EW_BLK, 1), 0)
    o_ref[:] = jnp.where(eid < E, val, 0.0)


def _edge_weights(ea_pad, a1t, c1, a2t, c2):
    return pl.pallas_call(
        _ew_body,
        grid=(E_PAD // _EW_BLK,),
        in_specs=[
            pl.BlockSpec((_EW_BLK, 4), lambda i: (i, 0)),
            pl.BlockSpec((4, 128), lambda i: (0, 0)),
            pl.BlockSpec((1, 128), lambda i: (0, 0)),
            pl.BlockSpec((128, D * D), lambda i: (0, 0)),
            pl.BlockSpec((1, D * D), lambda i: (0, 0)),
        ],
        out_specs=pl.BlockSpec((_EW_BLK, D * D), lambda i: (i, 0)),
        out_shape=jax.ShapeDtypeStruct((E_PAD, D * D), F32),
    )(ea_pad, a1t, c1.reshape(1, 128), a2t, c2.reshape(1, D * D))


_MSG_BLK = 512


def _msg_body(xj_ref, ew_ref, o_ref):
    acc = xj_ref[:, 0:1] * ew_ref[:, 0, :]
    for i in range(1, D):
        acc = acc + xj_ref[:, i:i + 1] * ew_ref[:, i, :]
    o_ref[:, 0:D] = acc
    o_ref[:, D:DW] = jnp.zeros((_MSG_BLK, DW - D), F32)


def _messages(xj, ew3):
    return pl.pallas_call(
        _msg_body,
        grid=(E_PAD // _MSG_BLK,),
        in_specs=[
            pl.BlockSpec((_MSG_BLK, DW), lambda i: (i, 0)),
            pl.BlockSpec((_MSG_BLK, D, D), lambda i: (i, 0, 0)),
        ],
        out_specs=pl.BlockSpec((_MSG_BLK, DW), lambda i: (i, 0)),
        out_shape=jax.ShapeDtypeStruct((E_PAD, DW), F32),
    )(xj, ew3)


_GRU_BLK = 512


def _gru_body(p0_ref, p1_ref, d0_ref, d1_ref, h_ref, rootT_ref, cb_ref,
              wr_ref, wz_ref, wn_ref, hr_ref, hz_ref, hn_ref,
              br_ref, bz_ref, bn_ref, o_ref):
    deg = jnp.maximum(d0_ref[:, 0:D] + d1_ref[:, 0:D], 1.0)
    agg = (p0_ref[:, 0:D] + p1_ref[:, 0:D]) / deg
    h = h_ref[:, 0:D]
    conv = agg + jnp.dot(h, rootT_ref[:], preferred_element_type=F32) + cb_ref[:]
    m = _lrelu(conv)
    r = jax.nn.sigmoid(jnp.dot(m, wr_ref[:], preferred_element_type=F32)
                       + jnp.dot(h, hr_ref[:], preferred_element_type=F32)
                       + br_ref[:])
    z = jax.nn.sigmoid(jnp.dot(m, wz_ref[:], preferred_element_type=F32)
                       + jnp.dot(h, hz_ref[:], preferred_element_type=F32)
                       + bz_ref[:])
    n = jnp.tanh(jnp.dot(m, wn_ref[:], preferred_element_type=F32)
                 + r * (jnp.dot(h, hn_ref[:], preferred_element_type=F32)
                        + bn_ref[:]))
    o_ref[:, 0:D] = (1.0 - z) * n + z * h
    o_ref[:, D:DW] = jnp.zeros((_GRU_BLK, DW - D), F32)


def _gru_update(p0, p1, d0, d1, h, rootT, cbias, gw):
    blk = lambda: pl.BlockSpec((_GRU_BLK, DW), lambda i: (i, 0))
    wspec = lambda: pl.BlockSpec((D, D), lambda i: (0, 0))
    bspec = lambda: pl.BlockSpec((1, D), lambda i: (0, 0))
    return pl.pallas_call(
        _gru_body,
        grid=(N_PAD // _GRU_BLK,),
        in_specs=[blk(), blk(), blk(), blk(), blk(), wspec(), bspec(),
                  wspec(), wspec(), wspec(), wspec(), wspec(), wspec(),
                  bspec(), bspec(), bspec()],
        out_specs=blk(),
        out_shape=jax.ShapeDtypeStruct((N_PAD, DW), F32),
    )(p0, p1, d0, d1, h, rootT, cbias, *gw)


def _s2s_body(h_ref, batch_ref, wq_ref, wr_ref, wh_ref, bias_ref,
              w3q_ref, w3r_ref, b3_ref, o_ref):
    hn = h_ref[:, 0:D]
    bid = batch_ref[:]
    gid = lax.broadcasted_iota(jnp.int32, (1, B), 1)
    rid = lax.broadcasted_iota(jnp.int32, (N_PAD, 1), 0)
    valid = rid < N
    mb = jnp.logical_and(bid == gid, valid)
    mf = mb.astype(F32)

    q = jnp.zeros((B, D), F32)
    rr = jnp.zeros((B, D), F32)
    hl = jnp.zeros((B, D), F32)
    cl = jnp.zeros((B, D), F32)
    dn = (((0,), (0,)), ((), ()))
    for _ in range(3):
        gates = []
        for g in range(4):
            gates.append(
                jnp.dot(q, wq_ref[g], preferred_element_type=F32)
                + jnp.dot(rr, wr_ref[g], preferred_element_type=F32)
                + jnp.dot(hl, wh_ref[g], preferred_element_type=F32)
                + bias_ref[g])
        ii = jax.nn.sigmoid(gates[0])
        ff = jax.nn.sigmoid(gates[1])
        gg = jnp.tanh(gates[2])
        oo = jax.nn.sigmoid(gates[3])
        cl = ff * cl + ii * gg
        hl = oo * jnp.tanh(cl)
        q = hl
        qb = jnp.dot(mf, q, preferred_element_type=F32)
        e = jnp.sum(hn * qb, axis=1, keepdims=True)
        em = jnp.where(mb, e, -1e30)
        emax = jnp.max(em, axis=0, keepdims=True)
        emax_n = lax.dot_general(mf, emax,
                                 (((1,), (1,)), ((), ())),
                                 preferred_element_type=F32)
        ee = jnp.where(valid, jnp.exp(e - emax_n), 0.0)
        esum = lax.dot_general(mf, ee, dn,
                               preferred_element_type=F32)
        esum_n = jnp.dot(mf, esum, preferred_element_type=F32)
        a = ee / (esum_n + 1e-16)
        rr = lax.dot_general(mf, a * hn, dn,
                             preferred_element_type=F32)
    val = (jnp.sum(q * w3q_ref[:], axis=1, keepdims=True)
           + jnp.sum(rr * w3r_ref[:], axis=1, keepdims=True)
           + b3_ref[:])
    o_ref[:] = val


def _set2set(h, batch2d, wq, wr, wh, bias, w3q, w3r, b3):
    return pl.pallas_call(
        _s2s_body,
        out_shape=jax.ShapeDtypeStruct((B, 1), F32),
    )(h, batch2d, wq, wr, wh, bias, w3q, w3r, b3)


def _make_sc_kernels():
    info = plsc.get_sparse_core_info()
    nc, ns = info.num_cores, info.num_subcores
    nw = nc * ns
    epw = E_PAD // nw
    nch = epw // CH
    mesh = plsc.VectorSubcoreMesh(core_axis_name="c", subcore_axis_name="s")
    rows_per_tile = N_PAD // ns

    @functools.partial(
        pl.kernel, mesh=mesh,
        out_type=jax.ShapeDtypeStruct((nw, nch, CH, DW), F32),
        scratch_types=[
            pltpu.VMEM((nch, CH), jnp.int32),
            pltpu.VMEM((nch, CH, DW), F32),
            pltpu.SemaphoreType.DMA,
        ],
    )
    def sc_gather(nodes_hbm, idx_hbm, xj_hbm, idx_v, rows_v, sem):
        c = lax.axis_index("c")
        s = lax.axis_index("s")
        wid = s * nc + c
        pltpu.sync_copy(idx_hbm.at[wid], idx_v)
        cps = [pltpu.async_copy(nodes_hbm.at[idx_v.at[j]], rows_v.at[j], sem)
               for j in range(nch)]
        for cp in cps:
            cp.wait()
        pltpu.sync_copy(rows_v, xj_hbm.at[wid])

    @functools.partial(
        pl.kernel, mesh=mesh,
        out_type=jax.ShapeDtypeStruct((nc, N_PAD, DW), F32),
        scratch_types=[
            pltpu.VMEM((nch, CH), jnp.int32),
            pltpu.VMEM((nch, CH, DW), F32),
            pltpu.VMEM_SHARED((N_PAD, DW), F32),
        ],
    )
    def sc_scatter_add(msg_hbm, idx_hbm, zero_hbm, part_hbm,
                       idx_v, rows_v, acc_sh):
        c = lax.axis_index("c")
        s = lax.axis_index("s")
        wid = s * nc + c
        row0 = s * rows_per_tile
        pltpu.sync_copy(zero_hbm.at[pl.ds(row0, rows_per_tile)],
                        acc_sh.at[pl.ds(row0, rows_per_tile)])
        plsc.subcore_barrier()
        pltpu.sync_copy(idx_hbm.at[wid], idx_v)
        pltpu.sync_copy(msg_hbm.at[wid], rows_v)
        for j in range(nch):
            pltpu.sync_copy(rows_v.at[j], acc_sh.at[idx_v.at[j]], add=True)
        plsc.subcore_barrier()
        pltpu.sync_copy(acc_sh.at[pl.ds(row0, rows_per_tile)],
                        part_hbm.at[c, pl.ds(row0, rows_per_tile)])

    return sc_gather, sc_scatter_add, nw, nch


def kernel(x, edge_attr, W0, b0, A1, c1, A2, c2, root, conv_bias,
           gru_wih, gru_whh, gru_bih, gru_bhh,
           lstm_wih, lstm_whh, lstm_bih, lstm_bhh, W3, b3,
           edge_index, batch):
    sc_gather, sc_scatter_add, nw, nch = _make_sc_kernels()

    x_pad = jnp.pad(x, ((0, N_PAD - N), (0, 0)))
    ea_pad = jnp.pad(edge_attr, ((0, E_PAD - E), (0, 0)))
    src_p = jnp.pad(edge_index[0], (0, E_PAD - E))
    dst_p = jnp.pad(edge_index[1], (0, E_PAD - E), constant_values=N)
    src_r = src_p.reshape(nw, nch, CH)
    dst_r = dst_p.reshape(nw, nch, CH)
    batch2d = jnp.pad(batch, (0, N_PAD - N)).reshape(N_PAD, 1)

    w0t = W0.T
    a1t = A1.T
    a2t = A2.T
    rootT = root
    cbias = conv_bias.reshape(1, D)
    gw = []
    for k in range(3):
        gw.append(gru_wih[k * D:(k + 1) * D].T)
    for k in range(3):
        gw.append(gru_whh[k * D:(k + 1) * D].T)
    for k in range(3):
        gw.append((gru_bih[k * D:(k + 1) * D]
                   + gru_bhh[k * D:(k + 1) * D]).reshape(1, D))
    wq = jnp.stack([lstm_wih[g * D:(g + 1) * D, :D].T for g in range(4)])
    wr = jnp.stack([lstm_wih[g * D:(g + 1) * D, D:].T for g in range(4)])
    wh = jnp.stack([lstm_whh[g * D:(g + 1) * D].T for g in range(4)])
    lbias = jnp.stack([(lstm_bih[g * D:(g + 1) * D]
                        + lstm_bhh[g * D:(g + 1) * D]).reshape(1, D)
                       for g in range(4)])
    w3q = W3[:, :D]
    w3r = W3[:, D:]
    b3_2d = b3.reshape(1, 1)
    zeros_nd = jnp.zeros((N_PAD, DW), F32)
    ones_msg = jnp.ones((nw, nch, CH, DW), F32)

    h = _lin0(x_pad, w0t, b0)
    ew = _edge_weights(ea_pad, a1t, c1, a2t, c2)
    ew3 = ew.reshape(E_PAD, D, D)

    deg_part = sc_scatter_add(ones_msg, dst_r, zeros_nd)
    d0, d1 = deg_part[0], deg_part[1]

    for _ in range(STEPS):
        xj = sc_gather(h, src_r)
        msg = _messages(xj.reshape(E_PAD, DW), ew3)
        part = sc_scatter_add(msg.reshape(nw, nch, CH, DW), dst_r, zeros_nd)
        h = _gru_update(part[0], part[1], d0, d1, h, rootT, cbias, gw)

    val = _set2set(h, batch2d, wq, wr, wh, lbias, w3q, w3r, b3_2d)
    return (val.reshape(B), h[:N, :D])


# speedup vs baseline: 3.9317x; 3.9317x over previous
import functools

import jax
import jax.numpy as jnp
from jax import lax
from jax.experimental import pallas as pl
from jax.experimental.pallas import tpu as pltpu
from jax.experimental.pallas import tpu_sc as plsc

N = 5000
E = 20000
FIN = 14
D = 64
DW = 128
B = 128
STEPS = 6

N_PAD = 5120
E_PAD = 20480
CH = 128

F32 = jnp.float32


def _lrelu(v):
    return jnp.where(v >= 0, v, 0.01 * v)


def _lin0_body(x_ref, w_ref, b_ref, o_ref):
    o_ref[:, 0:D] = _lrelu(
        jnp.dot(x_ref[:], w_ref[:], preferred_element_type=F32) + b_ref[:])
    o_ref[:, D:DW] = jnp.zeros((N_PAD, DW - D), F32)


def _lin0(x_pad, w0t, b0):
    return pl.pallas_call(
        _lin0_body,
        out_shape=jax.ShapeDtypeStruct((N_PAD, DW), F32),
    )(x_pad, w0t, b0.reshape(1, D))


_EW_BLK = 256


def _ew_body(eat_ref, a1aug_ref, a2aug_ref, o_ref):
    pid = pl.program_id(0)
    eat_aug = jnp.concatenate(
        [eat_ref[:], jnp.ones((1, _EW_BLK), F32)], axis=0)
    hidT = _lrelu(jnp.dot(a1aug_ref[:], eat_aug,
                          preferred_element_type=F32))
    hidT_aug = jnp.concatenate(
        [hidT, jnp.ones((1, _EW_BLK), F32)], axis=0)
    val = jnp.dot(a2aug_ref[:], hidT_aug,
                  preferred_element_type=F32)
    eid = pid * _EW_BLK + lax.broadcasted_iota(jnp.int32, (1, _EW_BLK), 1)
    o_ref[:] = jnp.where(eid < E, val, 0.0).astype(jnp.bfloat16)


def _edge_weights(eat_pad, a1aug, a2aug):
    return pl.pallas_call(
        _ew_body,
        grid=(E_PAD // _EW_BLK,),
        in_specs=[
            pl.BlockSpec((4, _EW_BLK), lambda i: (0, i)),
            pl.BlockSpec((128, 5), lambda i: (0, 0)),
            pl.BlockSpec((D * D, 129), lambda i: (0, 0)),
        ],
        out_specs=pl.BlockSpec((D * D, _EW_BLK), lambda i: (0, i)),
        out_shape=jax.ShapeDtypeStruct((D * D, E_PAD), jnp.bfloat16),
    )(eat_pad, a1aug, a2aug)


_MSG_BLK = 256


def _msg_body(xj_ref, ewt_ref, o_ref):
    xjt = jnp.swapaxes(xj_ref[:, 0:D], 0, 1)
    acc = jnp.zeros((D, _MSG_BLK), F32)
    for i in range(D):
        ewi = ewt_ref[i * D:(i + 1) * D, :].astype(F32)
        acc = acc + xjt[i:i + 1, :] * ewi
    o_ref[:, 0:D] = jnp.swapaxes(acc, 0, 1)
    o_ref[:, D:DW] = jnp.zeros((_MSG_BLK, DW - D), F32)


def _messages(xj, ewt):
    return pl.pallas_call(
        _msg_body,
        grid=(E_PAD // _MSG_BLK,),
        in_specs=[
            pl.BlockSpec((_MSG_BLK, DW), lambda i: (i, 0)),
            pl.BlockSpec((D * D, _MSG_BLK), lambda i: (0, i)),
        ],
        out_specs=pl.BlockSpec((_MSG_BLK, DW), lambda i: (i, 0)),
        out_shape=jax.ShapeDtypeStruct((E_PAD, DW), F32),
    )(xj, ewt)


_GRU_BLK = 512


def _gru_body(p0_ref, p1_ref, d0_ref, d1_ref, h_ref, rootT_ref, cb_ref,
              wr_ref, wz_ref, wn_ref, hr_ref, hz_ref, hn_ref,
              br_ref, bz_ref, bn_ref, o_ref):
    deg = jnp.maximum(d0_ref[:, 0:D] + d1_ref[:, 0:D], 1.0)
    agg = (p0_ref[:, 0:D] + p1_ref[:, 0:D]) / deg
    h = h_ref[:, 0:D]
    conv = agg + jnp.dot(h, rootT_ref[:], preferred_element_type=F32) + cb_ref[:]
    m = _lrelu(conv)
    r = jax.nn.sigmoid(jnp.dot(m, wr_ref[:], preferred_element_type=F32)
                       + jnp.dot(h, hr_ref[:], preferred_element_type=F32)
                       + br_ref[:])
    z = jax.nn.sigmoid(jnp.dot(m, wz_ref[:], preferred_element_type=F32)
                       + jnp.dot(h, hz_ref[:], preferred_element_type=F32)
                       + bz_ref[:])
    n = jnp.tanh(jnp.dot(m, wn_ref[:], preferred_element_type=F32)
                 + r * (jnp.dot(h, hn_ref[:], preferred_element_type=F32)
                        + bn_ref[:]))
    o_ref[:, 0:D] = (1.0 - z) * n + z * h
    o_ref[:, D:DW] = jnp.zeros((_GRU_BLK, DW - D), F32)


def _gru_update(p0, p1, d0, d1, h, rootT, cbias, gw):
    blk = lambda: pl.BlockSpec((_GRU_BLK, DW), lambda i: (i, 0))
    wspec = lambda: pl.BlockSpec((D, D), lambda i: (0, 0))
    bspec = lambda: pl.BlockSpec((1, D), lambda i: (0, 0))
    return pl.pallas_call(
        _gru_body,
        grid=(N_PAD // _GRU_BLK,),
        in_specs=[blk(), blk(), blk(), blk(), blk(), wspec(), bspec(),
                  wspec(), wspec(), wspec(), wspec(), wspec(), wspec(),
                  bspec(), bspec(), bspec()],
        out_specs=blk(),
        out_shape=jax.ShapeDtypeStruct((N_PAD, DW), F32),
    )(p0, p1, d0, d1, h, rootT, cbias, *gw)


def _s2s_body(h_ref, batch_ref, wq_ref, wr_ref, wh_ref, bias_ref,
              w3q_ref, w3r_ref, b3_ref, o_ref):
    hn = h_ref[:, 0:D]
    bid = batch_ref[:]
    gid = lax.broadcasted_iota(jnp.int32, (1, B), 1)
    rid = lax.broadcasted_iota(jnp.int32, (N_PAD, 1), 0)
    valid = rid < N
    mb = jnp.logical_and(bid == gid, valid)
    mf = mb.astype(F32)

    q = jnp.zeros((B, D), F32)
    rr = jnp.zeros((B, D), F32)
    hl = jnp.zeros((B, D), F32)
    cl = jnp.zeros((B, D), F32)
    dn = (((0,), (0,)), ((), ()))
    for _ in range(3):
        gates = []
        for g in range(4):
            gates.append(
                jnp.dot(q, wq_ref[g], preferred_element_type=F32)
                + jnp.dot(rr, wr_ref[g], preferred_element_type=F32)
                + jnp.dot(hl, wh_ref[g], preferred_element_type=F32)
                + bias_ref[g])
        ii = jax.nn.sigmoid(gates[0])
        ff = jax.nn.sigmoid(gates[1])
        gg = jnp.tanh(gates[2])
        oo = jax.nn.sigmoid(gates[3])
        cl = ff * cl + ii * gg
        hl = oo * jnp.tanh(cl)
        q = hl
        qb = jnp.dot(mf, q, preferred_element_type=F32)
        e = jnp.sum(hn * qb, axis=1, keepdims=True)
        em = jnp.where(mb, e, -1e30)
        emax = jnp.max(em, axis=0, keepdims=True)
        emax_n = lax.dot_general(mf, emax,
                                 (((1,), (1,)), ((), ())),
                                 preferred_element_type=F32)
        ee = jnp.where(valid, jnp.exp(e - emax_n), 0.0)
        esum = lax.dot_general(mf, ee, dn,
                               preferred_element_type=F32)
        esum_n = jnp.dot(mf, esum, preferred_element_type=F32)
        a = ee / (esum_n + 1e-16)
        rr = lax.dot_general(mf, a * hn, dn,
                             preferred_element_type=F32)
    val = (jnp.sum(q * w3q_ref[:], axis=1, keepdims=True)
           + jnp.sum(rr * w3r_ref[:], axis=1, keepdims=True)
           + b3_ref[:])
    o_ref[:] = val


def _set2set(h, batch2d, wq, wr, wh, bias, w3q, w3r, b3):
    return pl.pallas_call(
        _s2s_body,
        out_shape=jax.ShapeDtypeStruct((B, 1), F32),
    )(h, batch2d, wq, wr, wh, bias, w3q, w3r, b3)


def _make_sc_kernels():
    info = plsc.get_sparse_core_info()
    nc, ns = info.num_cores, info.num_subcores
    nw = nc * ns
    epw = E_PAD // nw
    nch = epw // CH
    mesh = plsc.VectorSubcoreMesh(core_axis_name="c", subcore_axis_name="s")
    rows_per_tile = N_PAD // ns

    @functools.partial(
        pl.kernel, mesh=mesh,
        out_type=jax.ShapeDtypeStruct((nw, nch, CH, DW), F32),
        scratch_types=[
            pltpu.VMEM((nch, CH), jnp.int32),
            pltpu.VMEM((nch, CH, DW), F32),
            pltpu.SemaphoreType.DMA,
        ],
    )
    def sc_gather(nodes_hbm, idx_hbm, xj_hbm, idx_v, rows_v, sem):
        c = lax.axis_index("c")
        s = lax.axis_index("s")
        wid = s * nc + c
        pltpu.sync_copy(idx_hbm.at[wid], idx_v)
        cps = [pltpu.async_copy(nodes_hbm.at[idx_v.at[j]], rows_v.at[j], sem)
               for j in range(nch)]
        for cp in cps:
            cp.wait()
        pltpu.sync_copy(rows_v, xj_hbm.at[wid])

    @functools.partial(
        pl.kernel, mesh=mesh,
        out_type=jax.ShapeDtypeStruct((nc, N_PAD, DW), F32),
        scratch_types=[
            pltpu.VMEM((nch, CH), jnp.int32),
            pltpu.VMEM((nch, CH, DW), F32),
            pltpu.VMEM_SHARED((N_PAD, DW), F32),
        ],
    )
    def sc_scatter_add(msg_hbm, idx_hbm, zero_hbm, part_hbm,
                       idx_v, rows_v, acc_sh):
        c = lax.axis_index("c")
        s = lax.axis_index("s")
        wid = s * nc + c
        row0 = s * rows_per_tile
        pltpu.sync_copy(zero_hbm.at[pl.ds(row0, rows_per_tile)],
                        acc_sh.at[pl.ds(row0, rows_per_tile)])
        plsc.subcore_barrier()
        pltpu.sync_copy(idx_hbm.at[wid], idx_v)
        pltpu.sync_copy(msg_hbm.at[wid], rows_v)
        for j in range(nch):
            pltpu.sync_copy(rows_v.at[j], acc_sh.at[idx_v.at[j]], add=True)
        plsc.subcore_barrier()
        pltpu.sync_copy(acc_sh.at[pl.ds(row0, rows_per_tile)],
                        part_hbm.at[c, pl.ds(row0, rows_per_tile)])

    return sc_gather, sc_scatter_add, nw, nch


def kernel(x, edge_attr, W0, b0, A1, c1, A2, c2, root, conv_bias,
           gru_wih, gru_whh, gru_bih, gru_bhh,
           lstm_wih, lstm_whh, lstm_bih, lstm_bhh, W3, b3,
           edge_index, batch):
    sc_gather, sc_scatter_add, nw, nch = _make_sc_kernels()

    x_pad = jnp.pad(x, ((0, N_PAD - N), (0, 0)))
    src_p = jnp.pad(edge_index[0], (0, E_PAD - E))
    dst_p = jnp.pad(edge_index[1], (0, E_PAD - E), constant_values=N)
    src_r = src_p.reshape(nw, nch, CH)
    dst_r = dst_p.reshape(nw, nch, CH)
    batch2d = jnp.pad(batch, (0, N_PAD - N)).reshape(N_PAD, 1)

    w0t = W0.T
    eat_pad = jnp.pad(edge_attr, ((0, E_PAD - E), (0, 0))).T
    a1aug = jnp.concatenate([A1, c1[:, None]], axis=1)
    a2aug = jnp.concatenate([A2, c2[:, None]], axis=1)
    rootT = root
    cbias = conv_bias.reshape(1, D)
    gw = []
    for k in range(3):
        gw.append(gru_wih[k * D:(k + 1) * D].T)
    for k in range(3):
        gw.append(gru_whh[k * D:(k + 1) * D].T)
    for k in range(3):
        gw.append((gru_bih[k * D:(k + 1) * D]
                   + gru_bhh[k * D:(k + 1) * D]).reshape(1, D))
    wq = jnp.stack([lstm_wih[g * D:(g + 1) * D, :D].T for g in range(4)])
    wr = jnp.stack([lstm_wih[g * D:(g + 1) * D, D:].T for g in range(4)])
    wh = jnp.stack([lstm_whh[g * D:(g + 1) * D].T for g in range(4)])
    lbias = jnp.stack([(lstm_bih[g * D:(g + 1) * D]
                        + lstm_bhh[g * D:(g + 1) * D]).reshape(1, D)
                       for g in range(4)])
    w3q = W3[:, :D]
    w3r = W3[:, D:]
    b3_2d = b3.reshape(1, 1)
    zeros_nd = jnp.zeros((N_PAD, DW), F32)
    ones_msg = jnp.ones((nw, nch, CH, DW), F32)

    h = _lin0(x_pad, w0t, b0)
    ewt = _edge_weights(eat_pad, a1aug, a2aug)

    deg_part = sc_scatter_add(ones_msg, dst_r, zeros_nd)
    d0, d1 = deg_part[0], deg_part[1]

    for _ in range(STEPS):
        xj = sc_gather(h, src_r)
        msg = _messages(xj.reshape(E_PAD, DW), ewt)
        part = sc_scatter_add(msg.reshape(nw, nch, CH, DW), dst_r, zeros_nd)
        h = _gru_update(part[0], part[1], d0, d1, h, rootT, cbias, gw)

    val = _set2set(h, batch2d, wq, wr, wh, lbias, w3q, w3r, b3_2d)
    return (val.reshape(B), h[:N, :D])
